# baseline (speedup 1.0000x reference)
import functools

import jax
import jax.numpy as jnp
from jax import lax
from jax.experimental import pallas as pl
from jax.experimental.pallas import tpu as pltpu
from jax.experimental.pallas import tpu_sc as plsc

_D = 64
_R = 128
_NBUF = 8
_CH = 2048


def _pack_table(table):
    v = table.shape[0]
    grid = (v + _CH - 1) // _CH
    h = _CH // 2

    def body(tt_ref, out_ref):
        blk = tt_ref[...]
        r = lax.broadcasted_iota(jnp.int32, (_D, _D), 0)
        c = lax.broadcasted_iota(jnp.int32, (_D, _D), 1)
        eye = jnp.where(r == c, 1.0, 0.0).astype(jnp.float32)
        t = lax.dot_general(blk, eye, (((0,), (0,)), ((), ())),
                            precision=lax.Precision.HIGHEST,
                            preferred_element_type=jnp.float32)
        out_ref[:, 0:_D] = t[0:h]
        out_ref[:, _D:2 * _D] = t[h:_CH]

    return pl.pallas_call(
        body,
        grid=(grid,),
        in_specs=[pl.BlockSpec((_D, _CH), lambda i: (0, i))],
        out_specs=pl.BlockSpec((h, 128), lambda i: (i, 0)),
        out_shape=jax.ShapeDtypeStruct((grid * h, 128), jnp.float32),
    )(jnp.swapaxes(table, 0, 1))


@functools.lru_cache(maxsize=None)
def _make_gather(n_rows, g_per_w, n_l, n_m):
    info = plsc.get_sparse_core_info()
    nc = info.num_cores
    mesh = plsc.VectorSubcoreMesh(core_axis_name="c", subcore_axis_name="s")
    nrounds = g_per_w // _NBUF
    tail = g_per_w - nrounds * _NBUF

    @functools.partial(
        pl.kernel,
        out_type=jax.ShapeDtypeStruct((n_l, 8, n_m, 8, _R), jnp.float32),
        mesh=mesh,
        compiler_params=pltpu.CompilerParams(
            use_tc_tiling_on_sc=False, needs_layout_passes=False,
            disable_bounds_checks=True),
        scratch_types=[
            pltpu.VMEM((g_per_w, _R), jnp.int32),
            pltpu.VMEM((_NBUF, _R, _D), jnp.float32),
            pltpu.VMEM((2, 8, 8, _R), jnp.float32),
            pltpu.SemaphoreType.DMA((_NBUF,)),
            pltpu.SemaphoreType.DMA((2,)),
        ],
    )
    def k(jdx_hbm, tp_hbm, out_hbm, jdx_v, g_v, t_v, gsems, wsems):
        wid = lax.axis_index("s") * nc + lax.axis_index("c")
        pltpu.sync_copy(jdx_hbm.at[wid], jdx_v)
        u0 = wid * g_per_w
        lanes = lax.iota(jnp.int32, 16)

        def gather(t, b):
            pltpu.async_copy(tp_hbm.at[jdx_v.at[t]], g_v.at[b], gsems.at[b])

        def wait_gather(b):
            pltpu.make_async_copy(
                tp_hbm.at[jdx_v.at[0]], g_v.at[b], gsems.at[b]).wait()

        def write(t, tb):
            u = u0 + t
            pltpu.async_copy(
                t_v.at[tb], out_hbm.at[u // n_m, :, u % n_m], wsems.at[tb])

        def wait_write(tb):
            pltpu.make_async_copy(
                t_v.at[tb], out_hbm.at[0, :, 0], wsems.at[tb]).wait()

        def transpose_unit(b, tb):
            gb = g_v.at[b]

            @plsc.parallel_loop(0, _R // 16, 1)
            def rg_body(rg):
                r0 = rg * 16
                rvec = lanes + r0
                for dk in range(8):
                    for dr in range(8):
                        cvec = jnp.full((16,), 8 * dk + dr, jnp.int32)
                        vals = plsc.load_gather(gb, [rvec, cvec])
                        t_v[tb, dk, dr, pl.ds(r0, 16)] = vals

        for b in range(_NBUF):
            gather(b, b)

        def unit_body(t, carry):
            b = t % _NBUF
            tb = t % 2
            wait_gather(b)

            @pl.when(t >= 2)
            def _():
                wait_write(tb)

            transpose_unit(b, tb)

            @pl.when(t + _NBUF < g_per_w)
            def _():
                gather(t + _NBUF, b)

            write(t, tb)
            return carry

        lax.fori_loop(0, g_per_w, unit_body, 0)
        wait_write(0)
        wait_write(1)

    return k


def kernel(x, table):
    b, l = x.shape
    n = b * l
    info = plsc.get_sparse_core_info()
    nw = info.num_cores * info.num_subcores
    g_per_w = n // (nw * _R)
    n_m = b // _R

    tp = _pack_table(table)
    tp64 = tp.reshape(tp.shape[0] * 2, _D)

    xi = x.astype(jnp.int32)
    x3d = jnp.swapaxes(xi, 0, 1).reshape(nw, g_per_w, _R)
    jdx3 = ((x3d >> 11) << 11) | ((x3d & 1023) << 1) | ((x3d >> 10) & 1)

    fn = _make_gather(tp64.shape[0], g_per_w, l, n_m)
    out5 = fn(jdx3, tp64)
    return out5.transpose(2, 4, 0, 1, 3).reshape(b, l, _D)


# speedup vs baseline: 1.7061x; 1.7061x over previous
import functools

import jax
import jax.numpy as jnp
from jax import lax
from jax.experimental import pallas as pl
from jax.experimental.pallas import tpu as pltpu
from jax.experimental.pallas import tpu_sc as plsc

_D = 64
_R = 128
_NBUF = 8
_CH = 2048


def _pack_table(table):
    v = table.shape[0]
    grid = (v + _CH - 1) // _CH
    h = _CH // 2

    def body(tt_ref, out_ref):
        blk = tt_ref[...]
        t = jnp.transpose(blk, (1, 0))
        out_ref[:, 0:_D] = t[0:h]
        out_ref[:, _D:2 * _D] = t[h:_CH]

    return pl.pallas_call(
        body,
        grid=(grid,),
        in_specs=[pl.BlockSpec((_D, _CH), lambda i: (0, i))],
        out_specs=pl.BlockSpec((h, 128), lambda i: (i, 0)),
        out_shape=jax.ShapeDtypeStruct((grid * h, 128), jnp.float32),
    )(jnp.swapaxes(table, 0, 1))


@functools.lru_cache(maxsize=None)
def _make_gather(n_rows, g_per_w, n_l, n_m):
    info = plsc.get_sparse_core_info()
    nc = info.num_cores
    mesh = plsc.VectorSubcoreMesh(core_axis_name="c", subcore_axis_name="s")
    nrounds = g_per_w // _NBUF
    tail = g_per_w - nrounds * _NBUF

    @functools.partial(
        pl.kernel,
        out_type=jax.ShapeDtypeStruct((n_l, 8, n_m, 8, _R), jnp.float32),
        mesh=mesh,
        compiler_params=pltpu.CompilerParams(
            use_tc_tiling_on_sc=False, needs_layout_passes=False,
            disable_bounds_checks=True),
        scratch_types=[
            pltpu.VMEM((g_per_w, _R), jnp.int32),
            pltpu.VMEM((_NBUF, _R, _D), jnp.float32),
            pltpu.VMEM((2, 8, 8, _R), jnp.float32),
            pltpu.SemaphoreType.DMA((_NBUF,)),
            pltpu.SemaphoreType.DMA((2,)),
        ],
    )
    def k(jdx_hbm, tp_hbm, out_hbm, jdx_v, g_v, t_v, gsems, wsems):
        wid = lax.axis_index("s") * nc + lax.axis_index("c")
        pltpu.sync_copy(jdx_hbm.at[wid], jdx_v)
        u0 = wid * g_per_w
        lanes = lax.iota(jnp.int32, 16)

        def gather(t, b):
            pltpu.async_copy(tp_hbm.at[jdx_v.at[t]], g_v.at[b], gsems.at[b])

        def wait_gather(b):
            pltpu.make_async_copy(
                tp_hbm.at[jdx_v.at[0]], g_v.at[b], gsems.at[b]).wait()

        def write(t, tb):
            u = u0 + t
            pltpu.async_copy(
                t_v.at[tb], out_hbm.at[u // n_m, :, u % n_m], wsems.at[tb])

        def wait_write(tb):
            pltpu.make_async_copy(
                t_v.at[tb], out_hbm.at[0, :, 0], wsems.at[tb]).wait()

        def transpose_unit(b, tb):
            gb = g_v.at[b]
            tb3 = t_v.at[tb]

            @plsc.parallel_loop(0, _R // 16, 1)
            def rg_body(rg):
                r0 = rg * 16
                rvec = lanes + r0
                for dg in range(4):
                    d0 = dg * 16
                    for s in range(16):
                        dvec = d0 + ((lanes + s) & 15)
                        vals = plsc.load_gather(gb, [rvec, dvec])
                        plsc.store_scatter(
                            tb3, [dvec >> 3, dvec & 7, rvec], vals)

        for b in range(_NBUF):
            gather(b, b)

        def unit_body(t, carry):
            b = t % _NBUF
            tb = t % 2
            wait_gather(b)

            @pl.when(t >= 2)
            def _():
                wait_write(tb)

            transpose_unit(b, tb)

            @pl.when(t + _NBUF < g_per_w)
            def _():
                gather(t + _NBUF, b)

            write(t, tb)
            return carry

        lax.fori_loop(0, g_per_w, unit_body, 0)
        wait_write(0)
        wait_write(1)

    return k


def kernel(x, table):
    b, l = x.shape
    n = b * l
    info = plsc.get_sparse_core_info()
    nw = info.num_cores * info.num_subcores
    g_per_w = n // (nw * _R)
    n_m = b // _R

    tp = _pack_table(table)
    tp64 = tp.reshape(tp.shape[0] * 2, _D)

    xi = x.astype(jnp.int32)
    x3d = jnp.swapaxes(xi, 0, 1).reshape(nw, g_per_w, _R)
    jdx3 = ((x3d >> 11) << 11) | ((x3d & 1023) << 1) | ((x3d >> 10) & 1)

    fn = _make_gather(tp64.shape[0], g_per_w, l, n_m)
    out5 = fn(jdx3, tp64)
    return out5.transpose(2, 4, 0, 1, 3).reshape(b, l, _D)


# speedup vs baseline: 2.0156x; 1.1814x over previous
import functools

import jax
import jax.numpy as jnp
from jax import lax
from jax.experimental import pallas as pl
from jax.experimental.pallas import tpu as pltpu
from jax.experimental.pallas import tpu_sc as plsc

_D = 64
_R = 128
_NBUF = 8
_CH = 4096
_CB = _CH.bit_length() - 1


def _pack_table(table):
    v = table.shape[0]
    grid = (v + _CH - 1) // _CH
    h = _CH // 2

    def body(tt_ref, out_ref):
        blk = tt_ref[...]
        t = jnp.transpose(blk, (1, 0))
        out_ref[:, 0:_D] = t[0:h]
        out_ref[:, _D:2 * _D] = t[h:_CH]

    return pl.pallas_call(
        body,
        grid=(grid,),
        in_specs=[pl.BlockSpec((_D, _CH), lambda i: (0, i))],
        out_specs=pl.BlockSpec((h, 128), lambda i: (i, 0)),
        out_shape=jax.ShapeDtypeStruct((grid * h, 128), jnp.float32),
    )(jnp.swapaxes(table, 0, 1))


@functools.lru_cache(maxsize=None)
def _make_gather(n_rows, g_per_w, n_l, n_m):
    info = plsc.get_sparse_core_info()
    nc = info.num_cores
    mesh = plsc.VectorSubcoreMesh(core_axis_name="c", subcore_axis_name="s")
    nrounds = g_per_w // _NBUF
    tail = g_per_w - nrounds * _NBUF

    @functools.partial(
        pl.kernel,
        out_type=jax.ShapeDtypeStruct((n_l, 8, n_m, 8, _R), jnp.float32),
        mesh=mesh,
        compiler_params=pltpu.CompilerParams(
            use_tc_tiling_on_sc=False, needs_layout_passes=False,
            disable_bounds_checks=True),
        scratch_types=[
            pltpu.VMEM((g_per_w, _R), jnp.int32),
            pltpu.VMEM((_NBUF, _R, _D), jnp.float32),
            pltpu.VMEM((2, 8, 8, _R), jnp.float32),
            pltpu.SemaphoreType.DMA((_NBUF,)),
            pltpu.SemaphoreType.DMA((2,)),
        ],
    )
    def k(jdx_hbm, tp_hbm, out_hbm, jdx_v, g_v, t_v, gsems, wsems):
        wid = lax.axis_index("s") * nc + lax.axis_index("c")
        pltpu.sync_copy(jdx_hbm.at[wid], jdx_v)
        u0 = wid * g_per_w
        lanes = lax.iota(jnp.int32, 16)

        def gather(t, b):
            pltpu.async_copy(tp_hbm.at[jdx_v.at[t]], g_v.at[b], gsems.at[b])

        def wait_gather(b):
            pltpu.make_async_copy(
                tp_hbm.at[jdx_v.at[0]], g_v.at[b], gsems.at[b]).wait()

        def write(t, tb):
            u = u0 + t
            pltpu.async_copy(
                t_v.at[tb], out_hbm.at[u // n_m, :, u % n_m], wsems.at[tb])

        def wait_write(tb):
            pltpu.make_async_copy(
                t_v.at[tb], out_hbm.at[0, :, 0], wsems.at[tb]).wait()

        def transpose_unit(b, tb):
            gb = g_v.at[b]
            tb3 = t_v.at[tb]

            @plsc.parallel_loop(0, _R // 16, 1)
            def rg_body(rg):
                r0 = rg * 16
                rvec = lanes + r0
                for dg in range(4):
                    d0 = dg * 16
                    for s in range(16):
                        dvec = d0 + ((lanes + s) & 15)
                        vals = plsc.load_gather(gb, [rvec, dvec])
                        plsc.store_scatter(
                            tb3, [dvec >> 3, dvec & 7, rvec], vals)

        for b in range(_NBUF):
            gather(b, b)

        def unit_body(t, carry):
            b = t % _NBUF
            tb = t % 2
            wait_gather(b)

            @pl.when(t >= 2)
            def _():
                wait_write(tb)

            transpose_unit(b, tb)

            @pl.when(t + _NBUF < g_per_w)
            def _():
                gather(t + _NBUF, b)

            write(t, tb)
            return carry

        lax.fori_loop(0, g_per_w, unit_body, 0)
        wait_write(0)
        wait_write(1)

    return k


def kernel(x, table):
    b, l = x.shape
    n = b * l
    info = plsc.get_sparse_core_info()
    nw = info.num_cores * info.num_subcores
    g_per_w = n // (nw * _R)
    n_m = b // _R

    tp = _pack_table(table)
    tp64 = tp.reshape(tp.shape[0] * 2, _D)

    xi = x.astype(jnp.int32)
    x3d = jnp.swapaxes(xi, 0, 1).reshape(nw, g_per_w, _R)
    jdx3 = (((x3d >> _CB) << _CB) | ((x3d & (_CH // 2 - 1)) << 1)
            | ((x3d >> (_CB - 1)) & 1))

    fn = _make_gather(tp64.shape[0], g_per_w, l, n_m)
    out5 = fn(jdx3, tp64)
    return out5.transpose(2, 4, 0, 1, 3).reshape(b, l, _D)


# speedup vs baseline: 2.2292x; 1.1060x over previous
import functools

import jax
import jax.numpy as jnp
from jax import lax
from jax.experimental import pallas as pl
from jax.experimental.pallas import tpu as pltpu
from jax.experimental.pallas import tpu_sc as plsc

_D = 64
_R = 128
_NBUF = 8
_CH = 8192
_CB = _CH.bit_length() - 1


def _pack_table(table):
    v = table.shape[0]
    grid = (v + _CH - 1) // _CH
    h = _CH // 2

    def body(tt_ref, out_ref):
        blk = tt_ref[...]
        t = jnp.transpose(blk, (1, 0))
        out_ref[:, 0:_D] = t[0:h]
        out_ref[:, _D:2 * _D] = t[h:_CH]

    return pl.pallas_call(
        body,
        grid=(grid,),
        in_specs=[pl.BlockSpec((_D, _CH), lambda i: (0, i))],
        out_specs=pl.BlockSpec((h, 128), lambda i: (i, 0)),
        out_shape=jax.ShapeDtypeStruct((grid * h, 128), jnp.float32),
    )(jnp.swapaxes(table, 0, 1))


@functools.lru_cache(maxsize=None)
def _make_gather(n_rows, g_per_w, n_l, n_m):
    info = plsc.get_sparse_core_info()
    nc = info.num_cores
    mesh = plsc.VectorSubcoreMesh(core_axis_name="c", subcore_axis_name="s")
    nrounds = g_per_w // _NBUF
    tail = g_per_w - nrounds * _NBUF

    @functools.partial(
        pl.kernel,
        out_type=jax.ShapeDtypeStruct((n_l, 8, n_m, 8, _R), jnp.float32),
        mesh=mesh,
        compiler_params=pltpu.CompilerParams(
            use_tc_tiling_on_sc=False, needs_layout_passes=False,
            disable_bounds_checks=True),
        scratch_types=[
            pltpu.VMEM((g_per_w, _R), jnp.int32),
            pltpu.VMEM((_NBUF, _R, _D), jnp.float32),
            pltpu.VMEM((2, 8, 8, _R), jnp.float32),
            pltpu.SemaphoreType.DMA((_NBUF,)),
            pltpu.SemaphoreType.DMA((2,)),
        ],
    )
    def k(jdx_hbm, tp_hbm, out_hbm, jdx_v, g_v, t_v, gsems, wsems):
        wid = lax.axis_index("s") * nc + lax.axis_index("c")
        pltpu.sync_copy(jdx_hbm.at[wid], jdx_v)
        u0 = wid * g_per_w
        lanes = lax.iota(jnp.int32, 16)

        def gather(t, b):
            pltpu.async_copy(tp_hbm.at[jdx_v.at[t]], g_v.at[b], gsems.at[b])

        def wait_gather(b):
            pltpu.make_async_copy(
                tp_hbm.at[jdx_v.at[0]], g_v.at[b], gsems.at[b]).wait()

        def write(t, tb):
            u = u0 + t
            pltpu.async_copy(
                t_v.at[tb], out_hbm.at[u // n_m, :, u % n_m], wsems.at[tb])

        def wait_write(tb):
            pltpu.make_async_copy(
                t_v.at[tb], out_hbm.at[0, :, 0], wsems.at[tb]).wait()

        def transpose_unit(b, tb):
            gb = g_v.at[b]
            tb3 = t_v.at[tb]

            @plsc.parallel_loop(0, _R // 16, 1)
            def rg_body(rg):
                r0 = rg * 16
                rvec = lanes + r0
                for dg in range(4):
                    d0 = dg * 16
                    for s in range(16):
                        dvec = d0 + ((lanes + s) & 15)
                        vals = plsc.load_gather(gb, [rvec, dvec])
                        plsc.store_scatter(
                            tb3, [dvec >> 3, dvec & 7, rvec], vals)

        for b in range(_NBUF):
            gather(b, b)

        def unit_body(t, carry):
            b = t % _NBUF
            tb = t % 2
            wait_gather(b)

            @pl.when(t >= 2)
            def _():
                wait_write(tb)

            transpose_unit(b, tb)

            @pl.when(t + _NBUF < g_per_w)
            def _():
                gather(t + _NBUF, b)

            write(t, tb)
            return carry

        lax.fori_loop(0, g_per_w, unit_body, 0)
        wait_write(0)
        wait_write(1)

    return k


def kernel(x, table):
    b, l = x.shape
    n = b * l
    info = plsc.get_sparse_core_info()
    nw = info.num_cores * info.num_subcores
    g_per_w = n // (nw * _R)
    n_m = b // _R

    tp = _pack_table(table)
    tp64 = tp.reshape(tp.shape[0] * 2, _D)

    xi = x.astype(jnp.int32)
    x3d = jnp.swapaxes(xi, 0, 1).reshape(nw, g_per_w, _R)
    jdx3 = (((x3d >> _CB) << _CB) | ((x3d & (_CH // 2 - 1)) << 1)
            | ((x3d >> (_CB - 1)) & 1))

    fn = _make_gather(tp64.shape[0], g_per_w, l, n_m)
    out5 = fn(jdx3, tp64)
    return out5.transpose(2, 4, 0, 1, 3).reshape(b, l, _D)


# speedup vs baseline: 2.3272x; 1.0440x over previous
import functools

import jax
import jax.numpy as jnp
from jax import lax
from jax.experimental import pallas as pl
from jax.experimental.pallas import tpu as pltpu
from jax.experimental.pallas import tpu_sc as plsc

_D = 64
_R = 128
_NBUF = 8
_CH = 16384
_CB = _CH.bit_length() - 1


def _pack_table(table):
    v = table.shape[0]
    grid = (v + _CH - 1) // _CH
    h = _CH // 2

    def body(tt_ref, out_ref):
        blk = tt_ref[...]
        t = jnp.transpose(blk, (1, 0))
        out_ref[:, 0:_D] = t[0:h]
        out_ref[:, _D:2 * _D] = t[h:_CH]

    return pl.pallas_call(
        body,
        grid=(grid,),
        in_specs=[pl.BlockSpec((_D, _CH), lambda i: (0, i))],
        out_specs=pl.BlockSpec((h, 128), lambda i: (i, 0)),
        out_shape=jax.ShapeDtypeStruct((grid * h, 128), jnp.float32),
    )(jnp.swapaxes(table, 0, 1))


@functools.lru_cache(maxsize=None)
def _make_gather(n_rows, g_per_w, n_l, n_m):
    info = plsc.get_sparse_core_info()
    nc = info.num_cores
    mesh = plsc.VectorSubcoreMesh(core_axis_name="c", subcore_axis_name="s")
    nrounds = g_per_w // _NBUF
    tail = g_per_w - nrounds * _NBUF

    @functools.partial(
        pl.kernel,
        out_type=jax.ShapeDtypeStruct((n_l, 8, n_m, 8, _R), jnp.float32),
        mesh=mesh,
        compiler_params=pltpu.CompilerParams(
            use_tc_tiling_on_sc=False, needs_layout_passes=False,
            disable_bounds_checks=True),
        scratch_types=[
            pltpu.VMEM((g_per_w, _R), jnp.int32),
            pltpu.VMEM((_NBUF, _R, _D), jnp.float32),
            pltpu.VMEM((2, 8, 8, _R), jnp.float32),
            pltpu.SemaphoreType.DMA((_NBUF,)),
            pltpu.SemaphoreType.DMA((2,)),
        ],
    )
    def k(jdx_hbm, tp_hbm, out_hbm, jdx_v, g_v, t_v, gsems, wsems):
        wid = lax.axis_index("s") * nc + lax.axis_index("c")
        pltpu.sync_copy(jdx_hbm.at[wid], jdx_v)
        u0 = wid * g_per_w
        lanes = lax.iota(jnp.int32, 16)

        def gather(t, b):
            pltpu.async_copy(tp_hbm.at[jdx_v.at[t]], g_v.at[b], gsems.at[b])

        def wait_gather(b):
            pltpu.make_async_copy(
                tp_hbm.at[jdx_v.at[0]], g_v.at[b], gsems.at[b]).wait()

        def write(t, tb):
            u = u0 + t
            pltpu.async_copy(
                t_v.at[tb], out_hbm.at[u // n_m, :, u % n_m], wsems.at[tb])

        def wait_write(tb):
            pltpu.make_async_copy(
                t_v.at[tb], out_hbm.at[0, :, 0], wsems.at[tb]).wait()

        def transpose_unit(b, tb):
            gb = g_v.at[b]
            tb3 = t_v.at[tb]

            @plsc.parallel_loop(0, _R // 16, 1)
            def rg_body(rg):
                r0 = rg * 16
                rvec = lanes + r0
                for dg in range(4):
                    d0 = dg * 16
                    for s in range(16):
                        dvec = d0 + ((lanes + s) & 15)
                        vals = plsc.load_gather(gb, [rvec, dvec])
                        plsc.store_scatter(
                            tb3, [dvec >> 3, dvec & 7, rvec], vals)

        for b in range(_NBUF):
            gather(b, b)

        def unit_body(t, carry):
            b = t % _NBUF
            tb = t % 2
            wait_gather(b)

            @pl.when(t >= 2)
            def _():
                wait_write(tb)

            transpose_unit(b, tb)

            @pl.when(t + _NBUF < g_per_w)
            def _():
                gather(t + _NBUF, b)

            write(t, tb)
            return carry

        lax.fori_loop(0, g_per_w, unit_body, 0)
        wait_write(0)
        wait_write(1)

    return k


def kernel(x, table):
    b, l = x.shape
    n = b * l
    info = plsc.get_sparse_core_info()
    nw = info.num_cores * info.num_subcores
    g_per_w = n // (nw * _R)
    n_m = b // _R

    tp = _pack_table(table)
    tp64 = tp.reshape(tp.shape[0] * 2, _D)

    xi = x.astype(jnp.int32)
    x3d = jnp.swapaxes(xi, 0, 1).reshape(nw, g_per_w, _R)
    jdx3 = (((x3d >> _CB) << _CB) | ((x3d & (_CH // 2 - 1)) << 1)
            | ((x3d >> (_CB - 1)) & 1))

    fn = _make_gather(tp64.shape[0], g_per_w, l, n_m)
    out5 = fn(jdx3, tp64)
    return out5.transpose(2, 4, 0, 1, 3).reshape(b, l, _D)


# speedup vs baseline: 2.3995x; 1.0310x over previous
import functools

import jax
import jax.numpy as jnp
from jax import lax
from jax.experimental import pallas as pl
from jax.experimental.pallas import tpu as pltpu
from jax.experimental.pallas import tpu_sc as plsc

_D = 64
_R = 128
_NBUF = 8
_CH = 32768
_CB = _CH.bit_length() - 1


def _pack_table(table):
    v = table.shape[0]
    grid = (v + _CH - 1) // _CH
    h = _CH // 2

    def body(tt_ref, out_ref):
        blk = tt_ref[...]
        t = jnp.transpose(blk, (1, 0))
        out_ref[:, 0:_D] = t[0:h]
        out_ref[:, _D:2 * _D] = t[h:_CH]

    return pl.pallas_call(
        body,
        grid=(grid,),
        in_specs=[pl.BlockSpec((_D, _CH), lambda i: (0, i))],
        out_specs=pl.BlockSpec((h, 128), lambda i: (i, 0)),
        out_shape=jax.ShapeDtypeStruct((grid * h, 128), jnp.float32),
    )(jnp.swapaxes(table, 0, 1))


@functools.lru_cache(maxsize=None)
def _make_gather(n_rows, g_per_w, n_l, n_m):
    info = plsc.get_sparse_core_info()
    nc = info.num_cores
    mesh = plsc.VectorSubcoreMesh(core_axis_name="c", subcore_axis_name="s")
    nrounds = g_per_w // _NBUF
    tail = g_per_w - nrounds * _NBUF

    @functools.partial(
        pl.kernel,
        out_type=jax.ShapeDtypeStruct((n_l, 8, n_m, 8, _R), jnp.float32),
        mesh=mesh,
        compiler_params=pltpu.CompilerParams(
            use_tc_tiling_on_sc=False, needs_layout_passes=False,
            disable_bounds_checks=True),
        scratch_types=[
            pltpu.VMEM((g_per_w, _R), jnp.int32),
            pltpu.VMEM((_NBUF, _R, _D), jnp.float32),
            pltpu.VMEM((2, 8, 8, _R), jnp.float32),
            pltpu.SemaphoreType.DMA((_NBUF,)),
            pltpu.SemaphoreType.DMA((2,)),
        ],
    )
    def k(jdx_hbm, tp_hbm, out_hbm, jdx_v, g_v, t_v, gsems, wsems):
        wid = lax.axis_index("s") * nc + lax.axis_index("c")
        pltpu.sync_copy(jdx_hbm.at[wid], jdx_v)
        u0 = wid * g_per_w
        lanes = lax.iota(jnp.int32, 16)

        def gather(t, b):
            pltpu.async_copy(tp_hbm.at[jdx_v.at[t]], g_v.at[b], gsems.at[b])

        def wait_gather(b):
            pltpu.make_async_copy(
                tp_hbm.at[jdx_v.at[0]], g_v.at[b], gsems.at[b]).wait()

        def write(t, tb):
            u = u0 + t
            pltpu.async_copy(
                t_v.at[tb], out_hbm.at[u // n_m, :, u % n_m], wsems.at[tb])

        def wait_write(tb):
            pltpu.make_async_copy(
                t_v.at[tb], out_hbm.at[0, :, 0], wsems.at[tb]).wait()

        def transpose_unit(b, tb):
            gb = g_v.at[b]
            tb3 = t_v.at[tb]

            @plsc.parallel_loop(0, _R // 16, 1)
            def rg_body(rg):
                r0 = rg * 16
                rvec = lanes + r0
                for dg in range(4):
                    d0 = dg * 16
                    for s in range(16):
                        dvec = d0 + ((lanes + s) & 15)
                        vals = plsc.load_gather(gb, [rvec, dvec])
                        plsc.store_scatter(
                            tb3, [dvec >> 3, dvec & 7, rvec], vals)

        for b in range(_NBUF):
            gather(b, b)

        def unit_body(t, carry):
            b = t % _NBUF
            tb = t % 2
            wait_gather(b)

            @pl.when(t >= 2)
            def _():
                wait_write(tb)

            transpose_unit(b, tb)

            @pl.when(t + _NBUF < g_per_w)
            def _():
                gather(t + _NBUF, b)

            write(t, tb)
            return carry

        lax.fori_loop(0, g_per_w, unit_body, 0)
        wait_write(0)
        wait_write(1)

    return k


def kernel(x, table):
    b, l = x.shape
    n = b * l
    info = plsc.get_sparse_core_info()
    nw = info.num_cores * info.num_subcores
    g_per_w = n // (nw * _R)
    n_m = b // _R

    tp = _pack_table(table)
    tp64 = tp.reshape(tp.shape[0] * 2, _D)

    xi = x.astype(jnp.int32)
    x3d = jnp.swapaxes(xi, 0, 1).reshape(nw, g_per_w, _R)
    jdx3 = (((x3d >> _CB) << _CB) | ((x3d & (_CH // 2 - 1)) << 1)
            | ((x3d >> (_CB - 1)) & 1))

    fn = _make_gather(tp64.shape[0], g_per_w, l, n_m)
    out5 = fn(jdx3, tp64)
    return out5.transpose(2, 4, 0, 1, 3).reshape(b, l, _D)
